# baseline (speedup 1.0000x reference)
import functools

import jax
import jax.numpy as jnp
from jax import lax
from jax.experimental import pallas as pl
from jax.experimental.pallas import tpu as pltpu
from jax.experimental.pallas import tpu_sc as plsc

_N = 10000
_E = 320000
_NC = 2
_NS = 16
_NW = _NC * _NS
_EPT = _E // _NW
_C = 80
_NCHUNK = _EPT // _C
_NP = 10240
_RPT = _NP // _NS
_ZR = 128


def _make_agg(D):
  mesh = plsc.VectorSubcoreMesh(core_axis_name="c", subcore_axis_name="s")

  @functools.partial(
      pl.kernel,
      mesh=mesh,
      out_type=jax.ShapeDtypeStruct((_NC * _NP, D), jnp.float32),
      scratch_types=[
          pltpu.VMEM((_EPT,), jnp.int32),
          pltpu.VMEM((_EPT,), jnp.int32),
          pltpu.VMEM((_C, D), jnp.float32),
          pltpu.VMEM((_C, D), jnp.float32),
          pltpu.VMEM_SHARED((_NP, D), jnp.float32),
          pltpu.SemaphoreType.DMA,
          pltpu.SemaphoreType.DMA,
          pltpu.SemaphoreType.DMA,
          pltpu.SemaphoreType.DMA,
          pltpu.SemaphoreType.DMA,
      ],
  )
  def agg(g_hbm, src_hbm, dst_hbm, zero_hbm, out_hbm,
          srcs_v, dsts_v, rows_a, rows_b, acc_sh,
          gsem_a, gsem_b, ssem_a, ssem_b, dsem):
    cid = lax.axis_index("c")
    sid = lax.axis_index("s")
    tid = cid * _NS + sid
    ebase = tid * _EPT

    pltpu.async_copy(src_hbm.at[pl.ds(ebase, _EPT)], srcs_v, dsem)
    pltpu.async_copy(dst_hbm.at[pl.ds(ebase, _EPT)], dsts_v, dsem)

    row0 = sid * _RPT
    for z in range(_RPT // _ZR):
      pltpu.sync_copy(zero_hbm, acc_sh.at[pl.ds(row0 + z * _ZR, _ZR)])

    pltpu.make_async_copy(src_hbm.at[pl.ds(ebase, _EPT)], srcs_v, dsem).wait()
    pltpu.make_async_copy(dst_hbm.at[pl.ds(ebase, _EPT)], dsts_v, dsem).wait()
    plsc.subcore_barrier()

    def sidx(k):
      return srcs_v.at[pl.ds(pl.multiple_of(k * _C, 8), _C)]

    def didx(k):
      return dsts_v.at[pl.ds(pl.multiple_of(k * _C, 8), _C)]

    pltpu.async_copy(g_hbm.at[sidx(0)], rows_a, gsem_a)

    def pair(p, carry):
      k = 2 * p
      pltpu.async_copy(g_hbm.at[sidx(k + 1)], rows_b, gsem_b)
      pltpu.make_async_copy(g_hbm.at[sidx(k)], rows_a, gsem_a).wait()
      pltpu.async_copy(rows_a, acc_sh.at[didx(k)], ssem_a, add=True)
      pltpu.make_async_copy(rows_a, acc_sh.at[didx(k)], ssem_a).wait()
      pltpu.async_copy(g_hbm.at[sidx(k + 2)], rows_a, gsem_a)
      pltpu.make_async_copy(g_hbm.at[sidx(k + 1)], rows_b, gsem_b).wait()
      pltpu.async_copy(rows_b, acc_sh.at[didx(k + 1)], ssem_b, add=True)
      pltpu.make_async_copy(rows_b, acc_sh.at[didx(k + 1)], ssem_b).wait()
      return carry

    lax.fori_loop(0, (_NCHUNK - 1) // 2, pair, 0)
    pltpu.make_async_copy(g_hbm.at[sidx(_NCHUNK - 1)], rows_a, gsem_a).wait()
    pltpu.sync_copy(rows_a, acc_sh.at[didx(_NCHUNK - 1)], add=True)

    plsc.subcore_barrier()
    pltpu.sync_copy(acc_sh.at[pl.ds(row0, _RPT)],
                    out_hbm.at[pl.ds(cid * _NP + row0, _RPT)])

  return agg


def _make_deg():
  mesh = plsc.VectorSubcoreMesh(core_axis_name="c", subcore_axis_name="s")
  _W = 16

  @functools.partial(
      pl.kernel,
      mesh=mesh,
      out_type=jax.ShapeDtypeStruct((_NC * _NP, 128), jnp.float32),
      scratch_types=[
          pltpu.VMEM((_EPT,), jnp.int32),
          pltpu.VMEM((_C, 128), jnp.float32),
          pltpu.VMEM_SHARED((_NP, 128), jnp.float32),
          pltpu.SemaphoreType.DMA,
          pltpu.SemaphoreType.DMA,
      ],
  )
  def deg(dst_hbm, ones_hbm, zero_hbm, out_hbm,
          dsts_v, ones_v, acc_sh, ssem, dsem):
    cid = lax.axis_index("c")
    sid = lax.axis_index("s")
    tid = cid * _NS + sid
    ebase = tid * _EPT

    pltpu.async_copy(dst_hbm.at[pl.ds(ebase, _EPT)], dsts_v, dsem)
    pltpu.sync_copy(ones_hbm, ones_v)
    row0 = sid * _RPT
    for z in range(_RPT // _ZR):
      pltpu.sync_copy(zero_hbm, acc_sh.at[pl.ds(row0 + z * _ZR, _ZR)])
    pltpu.make_async_copy(dst_hbm.at[pl.ds(ebase, _EPT)], dsts_v, dsem).wait()
    plsc.subcore_barrier()

    def didx(k):
      return dsts_v.at[pl.ds(pl.multiple_of(k * _C, 8), _C)]

    for k in range(_W):
      pltpu.async_copy(ones_v, acc_sh.at[didx(k)], ssem, add=True)

    def chunk(k, carry):
      pltpu.make_async_copy(ones_v, acc_sh.at[didx(0)], ssem).wait()
      pltpu.async_copy(ones_v, acc_sh.at[didx(k + _W)], ssem, add=True)
      return carry

    lax.fori_loop(0, _NCHUNK - _W, chunk, 0)
    for k in range(_W):
      pltpu.make_async_copy(ones_v, acc_sh.at[didx(0)], ssem).wait()

    plsc.subcore_barrier()
    pltpu.sync_copy(acc_sh.at[pl.ds(row0, _RPT)],
                    out_hbm.at[pl.ds(cid * _NP + row0, _RPT)])

  return deg


_agg128 = _make_agg(128)
_deg = _make_deg()


_R = 1000
_G = _N // _R


def _dinv_of(d0, d1):
  return lax.rsqrt(d0[:, 0:1] + d1[:, 0:1] + 1.0)


def _stage_in_body(x_ref, d0_ref, d1_ref, w_ref, g_ref):
  dinv = _dinv_of(d0_ref[...], d1_ref[...])
  h = jnp.dot(x_ref[...], w_ref[...], preferred_element_type=jnp.float32)
  g_ref[...] = h * dinv


_stage_in = pl.pallas_call(
    _stage_in_body,
    grid=(_G,),
    in_specs=[
        pl.BlockSpec((_R, 128), lambda i: (i, 0)),
        pl.BlockSpec((_R, 8), lambda i: (i, 0)),
        pl.BlockSpec((_R, 8), lambda i: (i, 0)),
        pl.BlockSpec((128, 128), lambda i: (0, 0)),
    ],
    out_specs=pl.BlockSpec((_R, 128), lambda i: (i, 0)),
    out_shape=jax.ShapeDtypeStruct((_N, 128), jnp.float32),
)


def _stage_mid_body(s0_ref, s1_ref, g_ref, d0_ref, d1_ref, b_ref, w_ref, o_ref):
  dinv = _dinv_of(d0_ref[...], d1_ref[...])
  t = dinv * (s0_ref[...] + s1_ref[...] + g_ref[...]) + b_ref[...]
  h = jnp.where(t >= 0.0, t, 0.01 * t)
  o_ref[...] = jnp.dot(h, w_ref[...], preferred_element_type=jnp.float32) * dinv


def _make_stage_mid(DO):
  return pl.pallas_call(
      _stage_mid_body,
      grid=(_G,),
      in_specs=[
          pl.BlockSpec((_R, 128), lambda i: (i, 0)),
          pl.BlockSpec((_R, 128), lambda i: (i, 0)),
          pl.BlockSpec((_R, 128), lambda i: (i, 0)),
          pl.BlockSpec((_R, 8), lambda i: (i, 0)),
          pl.BlockSpec((_R, 8), lambda i: (i, 0)),
          pl.BlockSpec((1, 128), lambda i: (0, 0)),
          pl.BlockSpec((128, DO), lambda i: (0, 0)),
      ],
      out_specs=pl.BlockSpec((_R, DO), lambda i: (i, 0)),
      out_shape=jax.ShapeDtypeStruct((_N, DO), jnp.float32),
  )


_stage_mid128 = _make_stage_mid(128)


def _stage_act_body(s0_ref, s1_ref, g_ref, d0_ref, d1_ref, b_ref, o_ref):
  dinv = _dinv_of(d0_ref[...], d1_ref[...])
  t = dinv * (s0_ref[...] + s1_ref[...] + g_ref[...]) + b_ref[...]
  h = jnp.where(t >= 0.0, t, 0.01 * t)
  o_ref[...] = h * dinv


_stage_act = pl.pallas_call(
    _stage_act_body,
    grid=(_G,),
    in_specs=[
        pl.BlockSpec((_R, 128), lambda i: (i, 0)),
        pl.BlockSpec((_R, 128), lambda i: (i, 0)),
        pl.BlockSpec((_R, 128), lambda i: (i, 0)),
        pl.BlockSpec((_R, 8), lambda i: (i, 0)),
        pl.BlockSpec((_R, 8), lambda i: (i, 0)),
        pl.BlockSpec((1, 128), lambda i: (0, 0)),
    ],
    out_specs=pl.BlockSpec((_R, 128), lambda i: (i, 0)),
    out_shape=jax.ShapeDtypeStruct((_N, 128), jnp.float32),
)


def _stage_out_body(s0_ref, s1_ref, g_ref, d0_ref, d1_ref, b_ref, w_ref, o_ref):
  dinv = _dinv_of(d0_ref[...], d1_ref[...])
  agg = dinv * (s0_ref[...] + s1_ref[...] + g_ref[...])
  t = jnp.dot(agg, w_ref[...], preferred_element_type=jnp.float32) + b_ref[...]
  m = jnp.max(t, axis=1, keepdims=True)
  e = jnp.exp(t - m)
  lse = jnp.log(jnp.sum(e, axis=1, keepdims=True)) + m
  o_ref[...] = t - lse


_stage_out = pl.pallas_call(
    _stage_out_body,
    grid=(_G,),
    in_specs=[
        pl.BlockSpec((_R, 128), lambda i: (i, 0)),
        pl.BlockSpec((_R, 128), lambda i: (i, 0)),
        pl.BlockSpec((_R, 128), lambda i: (i, 0)),
        pl.BlockSpec((_R, 8), lambda i: (i, 0)),
        pl.BlockSpec((_R, 8), lambda i: (i, 0)),
        pl.BlockSpec((1, 40), lambda i: (0, 0)),
        pl.BlockSpec((128, 40), lambda i: (0, 0)),
    ],
    out_specs=pl.BlockSpec((_R, 40), lambda i: (i, 0)),
    out_shape=jax.ShapeDtypeStruct((_N, 40), jnp.float32),
)


def kernel(x, edge_index, W1, b1, W2, b2, W3, b3):
  src = edge_index[0]
  dst = edge_index[1]
  ones128 = jnp.ones((_C, 128), jnp.float32)
  z128 = jnp.zeros((_ZR, 128), jnp.float32)

  degp = _deg(dst, ones128, z128)
  d0, d1 = degp[:_N, :8], degp[_NP:_NP + _N, :8]

  g1 = _stage_in(x, d0, d1, W1)
  s = _agg128(g1, src, dst, z128)
  g2 = _stage_mid128(s[:_N], s[_NP:_NP + _N], g1, d0, d1, b1.reshape(1, 128), W2)
  s = _agg128(g2, src, dst, z128)
  g3 = _stage_act(s[:_N], s[_NP:_NP + _N], g2, d0, d1, b2.reshape(1, 128))
  s = _agg128(g3, src, dst, z128)
  return _stage_out(s[:_N], s[_NP:_NP + _N], g3, d0, d1, b3.reshape(1, 40), W3)


# speedup vs baseline: 1.0073x; 1.0073x over previous
import functools

import jax
import jax.numpy as jnp
from jax import lax
from jax.experimental import pallas as pl
from jax.experimental.pallas import tpu as pltpu
from jax.experimental.pallas import tpu_sc as plsc

_N = 10000
_E = 320000
_NC = 2
_NS = 16
_NW = _NC * _NS
_EPT = _E // _NW
_C = 80
_NCHUNK = _EPT // _C
_NP = 10240
_RPT = _NP // _NS
_ZR = 128


def _make_agg(D):
  mesh = plsc.VectorSubcoreMesh(core_axis_name="c", subcore_axis_name="s")

  @functools.partial(
      pl.kernel,
      mesh=mesh,
      out_type=jax.ShapeDtypeStruct((_NC * _NP, D), jnp.float32),
      scratch_types=[
          pltpu.VMEM((_EPT,), jnp.int32),
          pltpu.VMEM((_EPT,), jnp.int32),
          pltpu.VMEM((_C, D), jnp.float32),
          pltpu.VMEM((_C, D), jnp.float32),
          pltpu.VMEM_SHARED((_NP, D), jnp.float32),
          pltpu.SemaphoreType.DMA,
          pltpu.SemaphoreType.DMA,
          pltpu.SemaphoreType.DMA,
          pltpu.SemaphoreType.DMA,
          pltpu.SemaphoreType.DMA,
          pltpu.SemaphoreType.DMA,
      ],
  )
  def agg(g_hbm, src_hbm, dst_hbm, zero_hbm, out_hbm,
          srcs_v, dsts_v, rows_a, rows_b, acc_sh,
          gsem_a, gsem_b, ssem_a, ssem_b, dsem, zsem):
    cid = lax.axis_index("c")
    sid = lax.axis_index("s")
    tid = cid * _NS + sid
    ebase = tid * _EPT

    pltpu.async_copy(src_hbm.at[pl.ds(ebase, _EPT)], srcs_v, dsem)
    pltpu.async_copy(dst_hbm.at[pl.ds(ebase, _EPT)], dsts_v, dsem)

    row0 = sid * _RPT
    for z in range(_RPT // _ZR):
      pltpu.async_copy(zero_hbm, acc_sh.at[pl.ds(row0 + z * _ZR, _ZR)], zsem)

    pltpu.make_async_copy(src_hbm.at[pl.ds(ebase, _EPT)], srcs_v, dsem).wait()
    pltpu.make_async_copy(dst_hbm.at[pl.ds(ebase, _EPT)], dsts_v, dsem).wait()

    def sidx(k):
      return srcs_v.at[pl.ds(pl.multiple_of(k * _C, 8), _C)]

    def didx(k):
      return dsts_v.at[pl.ds(pl.multiple_of(k * _C, 8), _C)]

    pltpu.async_copy(g_hbm.at[sidx(0)], rows_a, gsem_a)
    for z in range(_RPT // _ZR):
      pltpu.make_async_copy(zero_hbm, acc_sh.at[pl.ds(row0 + z * _ZR, _ZR)],
                            zsem).wait()
    plsc.subcore_barrier()

    def pair(p, carry):
      k = 2 * p
      pltpu.async_copy(g_hbm.at[sidx(k + 1)], rows_b, gsem_b)
      pltpu.make_async_copy(g_hbm.at[sidx(k)], rows_a, gsem_a).wait()
      pltpu.async_copy(rows_a, acc_sh.at[didx(k)], ssem_a, add=True)
      pltpu.make_async_copy(rows_a, acc_sh.at[didx(k)], ssem_a).wait()
      pltpu.async_copy(g_hbm.at[sidx(k + 2)], rows_a, gsem_a)
      pltpu.make_async_copy(g_hbm.at[sidx(k + 1)], rows_b, gsem_b).wait()
      pltpu.async_copy(rows_b, acc_sh.at[didx(k + 1)], ssem_b, add=True)
      pltpu.make_async_copy(rows_b, acc_sh.at[didx(k + 1)], ssem_b).wait()
      return carry

    lax.fori_loop(0, (_NCHUNK - 1) // 2, pair, 0)
    pltpu.make_async_copy(g_hbm.at[sidx(_NCHUNK - 1)], rows_a, gsem_a).wait()
    pltpu.sync_copy(rows_a, acc_sh.at[didx(_NCHUNK - 1)], add=True)

    plsc.subcore_barrier()
    pltpu.sync_copy(acc_sh.at[pl.ds(row0, _RPT)],
                    out_hbm.at[pl.ds(cid * _NP + row0, _RPT)])

  return agg


def _make_deg():
  mesh = plsc.VectorSubcoreMesh(core_axis_name="c", subcore_axis_name="s")
  _W = 16

  @functools.partial(
      pl.kernel,
      mesh=mesh,
      out_type=jax.ShapeDtypeStruct((_NC * _NP, 128), jnp.float32),
      scratch_types=[
          pltpu.VMEM((_EPT,), jnp.int32),
          pltpu.VMEM((_C, 128), jnp.float32),
          pltpu.VMEM_SHARED((_NP, 128), jnp.float32),
          pltpu.SemaphoreType.DMA,
          pltpu.SemaphoreType.DMA,
          pltpu.SemaphoreType.DMA,
      ],
  )
  def deg(dst_hbm, ones_hbm, zero_hbm, out_hbm,
          dsts_v, ones_v, acc_sh, ssem, dsem, zsem):
    cid = lax.axis_index("c")
    sid = lax.axis_index("s")
    tid = cid * _NS + sid
    ebase = tid * _EPT

    pltpu.async_copy(dst_hbm.at[pl.ds(ebase, _EPT)], dsts_v, dsem)
    row0 = sid * _RPT
    for z in range(_RPT // _ZR):
      pltpu.async_copy(zero_hbm, acc_sh.at[pl.ds(row0 + z * _ZR, _ZR)], zsem)
    pltpu.sync_copy(ones_hbm, ones_v)
    pltpu.make_async_copy(dst_hbm.at[pl.ds(ebase, _EPT)], dsts_v, dsem).wait()
    for z in range(_RPT // _ZR):
      pltpu.make_async_copy(zero_hbm, acc_sh.at[pl.ds(row0 + z * _ZR, _ZR)],
                            zsem).wait()
    plsc.subcore_barrier()

    def didx(k):
      return dsts_v.at[pl.ds(pl.multiple_of(k * _C, 8), _C)]

    for k in range(_W):
      pltpu.async_copy(ones_v, acc_sh.at[didx(k)], ssem, add=True)

    def chunk(k, carry):
      pltpu.make_async_copy(ones_v, acc_sh.at[didx(0)], ssem).wait()
      pltpu.async_copy(ones_v, acc_sh.at[didx(k + _W)], ssem, add=True)
      return carry

    lax.fori_loop(0, _NCHUNK - _W, chunk, 0)
    for k in range(_W):
      pltpu.make_async_copy(ones_v, acc_sh.at[didx(0)], ssem).wait()

    plsc.subcore_barrier()
    pltpu.sync_copy(acc_sh.at[pl.ds(row0, _RPT)],
                    out_hbm.at[pl.ds(cid * _NP + row0, _RPT)])

  return deg


_agg128 = _make_agg(128)
_deg = _make_deg()


_R = 1000
_G = _N // _R


def _dinv_of(d0, d1):
  return lax.rsqrt(d0[:, 0:1] + d1[:, 0:1] + 1.0)


def _stage_in_body(x_ref, d0_ref, d1_ref, w_ref, g_ref):
  dinv = _dinv_of(d0_ref[...], d1_ref[...])
  h = jnp.dot(x_ref[...], w_ref[...], preferred_element_type=jnp.float32)
  g_ref[...] = h * dinv


_stage_in = pl.pallas_call(
    _stage_in_body,
    grid=(_G,),
    in_specs=[
        pl.BlockSpec((_R, 128), lambda i: (i, 0)),
        pl.BlockSpec((_R, 8), lambda i: (i, 0)),
        pl.BlockSpec((_R, 8), lambda i: (i, 0)),
        pl.BlockSpec((128, 128), lambda i: (0, 0)),
    ],
    out_specs=pl.BlockSpec((_R, 128), lambda i: (i, 0)),
    out_shape=jax.ShapeDtypeStruct((_N, 128), jnp.float32),
)


def _stage_mid_body(s0_ref, s1_ref, g_ref, d0_ref, d1_ref, b_ref, w_ref, o_ref):
  dinv = _dinv_of(d0_ref[...], d1_ref[...])
  t = dinv * (s0_ref[...] + s1_ref[...] + g_ref[...]) + b_ref[...]
  h = jnp.where(t >= 0.0, t, 0.01 * t)
  o_ref[...] = jnp.dot(h, w_ref[...], preferred_element_type=jnp.float32) * dinv


def _make_stage_mid(DO):
  return pl.pallas_call(
      _stage_mid_body,
      grid=(_G,),
      in_specs=[
          pl.BlockSpec((_R, 128), lambda i: (i, 0)),
          pl.BlockSpec((_R, 128), lambda i: (i, 0)),
          pl.BlockSpec((_R, 128), lambda i: (i, 0)),
          pl.BlockSpec((_R, 8), lambda i: (i, 0)),
          pl.BlockSpec((_R, 8), lambda i: (i, 0)),
          pl.BlockSpec((1, 128), lambda i: (0, 0)),
          pl.BlockSpec((128, DO), lambda i: (0, 0)),
      ],
      out_specs=pl.BlockSpec((_R, DO), lambda i: (i, 0)),
      out_shape=jax.ShapeDtypeStruct((_N, DO), jnp.float32),
  )


_stage_mid128 = _make_stage_mid(128)


def _stage_act_body(s0_ref, s1_ref, g_ref, d0_ref, d1_ref, b_ref, o_ref):
  dinv = _dinv_of(d0_ref[...], d1_ref[...])
  t = dinv * (s0_ref[...] + s1_ref[...] + g_ref[...]) + b_ref[...]
  h = jnp.where(t >= 0.0, t, 0.01 * t)
  o_ref[...] = h * dinv


_stage_act = pl.pallas_call(
    _stage_act_body,
    grid=(_G,),
    in_specs=[
        pl.BlockSpec((_R, 128), lambda i: (i, 0)),
        pl.BlockSpec((_R, 128), lambda i: (i, 0)),
        pl.BlockSpec((_R, 128), lambda i: (i, 0)),
        pl.BlockSpec((_R, 8), lambda i: (i, 0)),
        pl.BlockSpec((_R, 8), lambda i: (i, 0)),
        pl.BlockSpec((1, 128), lambda i: (0, 0)),
    ],
    out_specs=pl.BlockSpec((_R, 128), lambda i: (i, 0)),
    out_shape=jax.ShapeDtypeStruct((_N, 128), jnp.float32),
)


def _stage_out_body(s0_ref, s1_ref, g_ref, d0_ref, d1_ref, b_ref, w_ref, o_ref):
  dinv = _dinv_of(d0_ref[...], d1_ref[...])
  agg = dinv * (s0_ref[...] + s1_ref[...] + g_ref[...])
  t = jnp.dot(agg, w_ref[...], preferred_element_type=jnp.float32) + b_ref[...]
  m = jnp.max(t, axis=1, keepdims=True)
  e = jnp.exp(t - m)
  lse = jnp.log(jnp.sum(e, axis=1, keepdims=True)) + m
  o_ref[...] = t - lse


_stage_out = pl.pallas_call(
    _stage_out_body,
    grid=(_G,),
    in_specs=[
        pl.BlockSpec((_R, 128), lambda i: (i, 0)),
        pl.BlockSpec((_R, 128), lambda i: (i, 0)),
        pl.BlockSpec((_R, 128), lambda i: (i, 0)),
        pl.BlockSpec((_R, 8), lambda i: (i, 0)),
        pl.BlockSpec((_R, 8), lambda i: (i, 0)),
        pl.BlockSpec((1, 40), lambda i: (0, 0)),
        pl.BlockSpec((128, 40), lambda i: (0, 0)),
    ],
    out_specs=pl.BlockSpec((_R, 40), lambda i: (i, 0)),
    out_shape=jax.ShapeDtypeStruct((_N, 40), jnp.float32),
)


def kernel(x, edge_index, W1, b1, W2, b2, W3, b3):
  src = edge_index[0]
  dst = edge_index[1]
  ones128 = jnp.ones((_C, 128), jnp.float32)
  z128 = jnp.zeros((_ZR, 128), jnp.float32)

  degp = _deg(dst, ones128, z128)
  d0, d1 = degp[:_N, :8], degp[_NP:_NP + _N, :8]

  g1 = _stage_in(x, d0, d1, W1)
  s = _agg128(g1, src, dst, z128)
  g2 = _stage_mid128(s[:_N], s[_NP:_NP + _N], g1, d0, d1, b1.reshape(1, 128), W2)
  s = _agg128(g2, src, dst, z128)
  g3 = _stage_act(s[:_N], s[_NP:_NP + _N], g2, d0, d1, b2.reshape(1, 128))
  s = _agg128(g3, src, dst, z128)
  return _stage_out(s[:_N], s[_NP:_NP + _N], g3, d0, d1, b3.reshape(1, 40), W3)
